# Initial kernel scaffold; baseline (speedup 1.0000x reference)
import jax
import jax.numpy as jnp
from jax.experimental import pallas as pl


def kernel(idx):
    raise NotImplementedError("write your pallas kernel here")


# speedup vs baseline: 1.3740x; 1.3740x over previous
import jax
import jax.numpy as jnp
from jax.experimental import pallas as pl

_NUM_CLASSES = 100000
_BLOCK_COLS = 1024


def _onehot_block(idx_ref, out_ref):
    j = pl.program_id(0)
    base = j * _BLOCK_COLS
    idx = idx_ref[:]
    b = idx.shape[0]
    cols = jax.lax.broadcasted_iota(jnp.int32, (b, _BLOCK_COLS), 1) + base
    out_ref[:, :] = (idx == cols).astype(jnp.float32)


def kernel(idx):
    b = idx.shape[0]
    idx2 = idx.astype(jnp.int32).reshape(b, 1)
    grid = (pl.cdiv(_NUM_CLASSES, _BLOCK_COLS),)
    return pl.pallas_call(
        _onehot_block,
        grid=grid,
        in_specs=[pl.BlockSpec((b, 1), lambda j: (0, 0))],
        out_specs=pl.BlockSpec((b, _BLOCK_COLS), lambda j: (0, j)),
        out_shape=jax.ShapeDtypeStruct((b, _NUM_CLASSES), jnp.float32),
    )(idx2)
